# baseline (speedup 1.0000x reference)
import functools
import jax
import jax.numpy as jnp
from jax import lax
from jax.experimental import pallas as pl
from jax.experimental.pallas import tpu as pltpu
from jax.experimental.pallas import tpu_sc as plsc

D = 64
SCALE = 8.0
G = 128
GPC = 4
CHUNK = G * GPC
NC = 2
NS = 16
NW = NC * NS


def _body(nchunks, x_hbm, table_hbm, out_hbm, idx_v, rows_v, gsem):
    wid = lax.axis_index("s") * NC + lax.axis_index("c")
    row0 = wid * (nchunks * GPC)
    out0 = wid * (nchunks * CHUNK)

    def chunk_body(c, carry):
        pltpu.sync_copy(x_hbm.at[pl.ds(row0 + c * GPC, GPC)], idx_v)
        copies = [
            pltpu.async_copy(
                table_hbm.at[idx_v.at[j]],
                rows_v.at[pl.ds(j * G, G)],
                gsem,
            )
            for j in range(GPC)
        ]
        for cp in copies:
            cp.wait()

        def scale_row(i, carry2):
            r = rows_v.at[i]
            for j in range(D // 16):
                r[pl.ds(j * 16, 16)] = r[pl.ds(j * 16, 16)] * SCALE
            return carry2

        lax.fori_loop(0, CHUNK, scale_row, 0, unroll=4)
        pltpu.sync_copy(rows_v, out_hbm.at[pl.ds(out0 + c * CHUNK, CHUNK)])
        return carry

    lax.fori_loop(0, nchunks, chunk_body, 0)


@functools.partial(jax.jit, static_argnames=("n",))
def _sc_lookup(xf, table, n):
    nchunks = n // (NW * CHUNK)
    mesh = plsc.VectorSubcoreMesh(core_axis_name="c", subcore_axis_name="s")
    k = pl.kernel(
        functools.partial(_body, nchunks),
        mesh=mesh,
        compiler_params=pltpu.CompilerParams(use_tc_tiling_on_sc=False),
        out_type=jax.ShapeDtypeStruct((n, D), jnp.float32),
        scratch_types=[
            pltpu.VMEM((GPC, G), jnp.int32),
            pltpu.VMEM((CHUNK, D), jnp.float32),
            pltpu.SemaphoreType.DMA,
        ],
    )
    return k(xf, table)


def kernel(x, table):
    n = x.size
    xf = x.reshape(n // G, G)
    out = _sc_lookup(xf, table, n)
    return out.reshape(*x.shape, D)
